# baseline (speedup 1.0000x reference)
import jax
import jax.numpy as jnp
from jax.experimental import pallas as pl
from jax.experimental.pallas import tpu as pltpu

N = 10000
D = 128
BM = 400
NBUF = 3
NSTEP = N // BM


def _gcn_kernel(an_hbm, x_ref, w_ref, b_ref, out_ref, buf, sems):
    i = pl.program_id(0)

    def start(j):
        slot = jax.lax.rem(j, NBUF)
        pltpu.make_async_copy(
            an_hbm.at[pl.ds(j * BM, BM), :], buf.at[slot], sems.at[slot]
        ).start()

    @pl.when(i == 0)
    def _prologue():
        for j in range(NBUF):
            start(j)

    @pl.when(jnp.logical_and(i > 0, i + NBUF - 1 < NSTEP))
    def _refill():
        start(i + NBUF - 1)

    slot = jax.lax.rem(i, NBUF)
    pltpu.make_async_copy(
        an_hbm.at[pl.ds(i * BM, BM), :], buf.at[slot], sems.at[slot]
    ).wait()
    t = jnp.dot(buf[slot], x_ref[...], preferred_element_type=jnp.float32)
    out_ref[...] = (
        jnp.dot(t, w_ref[...], preferred_element_type=jnp.float32) + b_ref[...]
    )


def kernel(An, X, weight, bias):
    bias2d = bias.reshape(1, D)
    return pl.pallas_call(
        _gcn_kernel,
        grid=(NSTEP,),
        in_specs=[
            pl.BlockSpec(memory_space=pltpu.HBM),
            pl.BlockSpec((N, D), lambda i: (0, 0)),
            pl.BlockSpec((D, D), lambda i: (0, 0)),
            pl.BlockSpec((1, D), lambda i: (0, 0)),
        ],
        out_specs=pl.BlockSpec((BM, D), lambda i: (i, 0)),
        out_shape=jax.ShapeDtypeStruct((N, D), jnp.float32),
        scratch_shapes=[
            pltpu.VMEM((NBUF, BM, N), jnp.float32),
            pltpu.SemaphoreType.DMA((NBUF,)),
        ],
        compiler_params=pltpu.CompilerParams(
            dimension_semantics=("arbitrary",),
        ),
    )(An, X, weight, bias2d)


# speedup vs baseline: 1.0010x; 1.0010x over previous
import jax
import jax.numpy as jnp
from jax.experimental import pallas as pl
from jax.experimental.pallas import tpu as pltpu

N = 10000
D = 128
BM = 200
NBUF = 6
NSTEP = N // BM


def _gcn_kernel(an_hbm, x_ref, w_ref, b_ref, out_ref, buf, sems):
    i = pl.program_id(0)

    def start(j):
        slot = jax.lax.rem(j, NBUF)
        pltpu.make_async_copy(
            an_hbm.at[pl.ds(j * BM, BM), :], buf.at[slot], sems.at[slot]
        ).start()

    @pl.when(i == 0)
    def _prologue():
        for j in range(NBUF):
            start(j)

    @pl.when(jnp.logical_and(i > 0, i + NBUF - 1 < NSTEP))
    def _refill():
        start(i + NBUF - 1)

    slot = jax.lax.rem(i, NBUF)
    pltpu.make_async_copy(
        an_hbm.at[pl.ds(i * BM, BM), :], buf.at[slot], sems.at[slot]
    ).wait()
    t = jnp.dot(buf[slot], x_ref[...], preferred_element_type=jnp.float32)
    out_ref[...] = (
        jnp.dot(t, w_ref[...], preferred_element_type=jnp.float32) + b_ref[...]
    )


def kernel(An, X, weight, bias):
    bias2d = bias.reshape(1, D)
    return pl.pallas_call(
        _gcn_kernel,
        grid=(NSTEP,),
        in_specs=[
            pl.BlockSpec(memory_space=pltpu.HBM),
            pl.BlockSpec((N, D), lambda i: (0, 0)),
            pl.BlockSpec((D, D), lambda i: (0, 0)),
            pl.BlockSpec((1, D), lambda i: (0, 0)),
        ],
        out_specs=pl.BlockSpec((BM, D), lambda i: (i, 0)),
        out_shape=jax.ShapeDtypeStruct((N, D), jnp.float32),
        scratch_shapes=[
            pltpu.VMEM((NBUF, BM, N), jnp.float32),
            pltpu.SemaphoreType.DMA((NBUF,)),
        ],
        compiler_params=pltpu.CompilerParams(
            dimension_semantics=("arbitrary",),
        ),
    )(An, X, weight, bias2d)


# speedup vs baseline: 1.0364x; 1.0354x over previous
import jax
import jax.numpy as jnp
from jax.experimental import pallas as pl
from jax.experimental.pallas import tpu as pltpu

N = 10000
D = 128
BM = 200


def _gcn_kernel(an_a, an_b, x_ref, w_ref, b_ref, out_ref):
    ta = jnp.dot(an_a[...], x_ref[...], preferred_element_type=jnp.float32)
    tb = jnp.dot(an_b[...], x_ref[...], preferred_element_type=jnp.float32)
    out_ref[:BM, :] = (
        jnp.dot(ta, w_ref[...], preferred_element_type=jnp.float32) + b_ref[...]
    )
    out_ref[BM:, :] = (
        jnp.dot(tb, w_ref[...], preferred_element_type=jnp.float32) + b_ref[...]
    )


def kernel(An, X, weight, bias):
    bias2d = bias.reshape(1, D)
    grid = (N // (2 * BM),)
    return pl.pallas_call(
        _gcn_kernel,
        grid=grid,
        in_specs=[
            pl.BlockSpec((BM, N), lambda i: (2 * i, 0)),
            pl.BlockSpec((BM, N), lambda i: (2 * i + 1, 0)),
            pl.BlockSpec((N, D), lambda i: (0, 0)),
            pl.BlockSpec((D, D), lambda i: (0, 0)),
            pl.BlockSpec((1, D), lambda i: (0, 0)),
        ],
        out_specs=pl.BlockSpec((2 * BM, D), lambda i: (i, 0)),
        out_shape=jax.ShapeDtypeStruct((N, D), jnp.float32),
        compiler_params=pltpu.CompilerParams(
            dimension_semantics=("arbitrary",),
        ),
    )(An, An, X, weight, bias2d)
